# baseline (speedup 1.0000x reference)
import functools

import jax
import jax.numpy as jnp
from jax import lax
from jax.experimental import pallas as pl
from jax.experimental.pallas import tpu as pltpu
from jax.experimental.pallas import tpu_sc as plsc

B = 4
N = 524288
S = 32
SEG = N // S
HH = 256
WW = 256
START = 2

NC = 2
NS = 16
NW = NC * NS
PAIRS = B * S
PER_W = PAIRS // NW

CHUNK = 4096
CH_W = CHUNK * 5
ROW_W = SEG * 5


def _hist_body(ev_hbm, out_hbm, hist_v, buf0, buf1, sem_a, sem_b, sem_out):
    cid = lax.axis_index("c")
    sid = lax.axis_index("s")
    wid = sid * NC + cid

    lane5 = jnp.arange(16, dtype=jnp.int32) * 5
    ones = jnp.ones((16,), jnp.float32)
    zeros = jnp.zeros((16,), jnp.float32)

    bufs = (buf0, buf1)
    sems = (sem_a, sem_b)
    nchunks = SEG // CHUNK
    out_cp = None
    for k in range(PER_W):
        p = wid * PER_W + k
        row = p * ROW_W
        cps = [None, None]
        cps[0] = pltpu.async_copy(
            ev_hbm.at[pl.ds(row, CH_W)], bufs[0], sems[0])
        if out_cp is not None:
            out_cp.wait()

        def _zero(r, _):
            base = r * 256
            for u in range(16):
                hist_v[pl.ds(base + u * 16, 16)] = zeros
            return 0
        lax.fori_loop(0, (HH * WW) // 256, _zero, 0)

        for ci in range(nchunks):
            cur = ci % 2
            if ci + 1 < nchunks:
                nxt = (ci + 1) % 2
                cps[nxt] = pltpu.async_copy(
                    ev_hbm.at[pl.ds(row + (ci + 1) * CH_W, CH_W)],
                    bufs[nxt], sems[nxt])
            cps[cur].wait()
            buf = bufs[cur]

            def _scat(j, _):
                base = j * 320
                for u in range(4):
                    off = lane5 + (base + u * 80)
                    x = plsc.load_gather(buf, [off])
                    y = plsc.load_gather(buf, [off + 1])
                    bins = (y * 256.0 + x).astype(jnp.int32)
                    plsc.addupdate_scatter(hist_v, [bins], ones)
                return 0
            lax.fori_loop(0, CHUNK // 64, _scat, 0)

        out_cp = pltpu.async_copy(hist_v, out_hbm.at[p], sem_out)
    out_cp.wait()


@functools.lru_cache(maxsize=1)
def _hist_sc():
    return pl.kernel(
        _hist_body,
        out_type=jax.ShapeDtypeStruct((PAIRS, HH * WW), jnp.float32),
        mesh=plsc.VectorSubcoreMesh(core_axis_name="c", subcore_axis_name="s"),
        compiler_params=pltpu.CompilerParams(needs_layout_passes=False),
        scratch_types=[
            pltpu.VMEM((HH * WW,), jnp.float32),
            pltpu.VMEM((CH_W,), jnp.float32),
            pltpu.VMEM((CH_W,), jnp.float32),
            pltpu.SemaphoreType.DMA,
            pltpu.SemaphoreType.DMA,
            pltpu.SemaphoreType.DMA,
        ],
    )


def _dot(a, b, prec=lax.Precision.DEFAULT):
    return lax.dot_general(
        a, b, (((1,), (0,)), ((), ())),
        precision=prec,
        preferred_element_type=jnp.float32)


def _bdot(a, b):
    return _dot(a.astype(jnp.bfloat16), b.astype(jnp.bfloat16))


def _split256(x):
    hi = jnp.floor(x * (1.0 / 256.0))
    lo = x - 256.0 * hi
    return hi.astype(jnp.bfloat16), lo.astype(jnp.bfloat16)


def _exact_lmul(m, x):
    hi, lo = _split256(x)
    mb = m.astype(jnp.bfloat16)
    return 256.0 * _dot(mb, hi) + _dot(mb, lo)


def _exact_rmul(x, m):
    hi, lo = _split256(x)
    mb = m.astype(jnp.bfloat16)
    return 256.0 * _dot(hi, mb) + _dot(lo, mb)


def _top2_noise(along):
    v = along.astype(jnp.int32)
    si = lax.broadcasted_iota(jnp.int32, (S, WW), 0)
    li = lax.broadcasted_iota(jnp.int32, (S, WW), 1)
    flat = si * WW + li
    key = v * 8192 + (8191 - flat)
    k1 = jnp.max(key)
    f1 = 8191 - (k1 & 8191)
    key2 = jnp.where(flat == f1, -1, key)
    k2 = jnp.max(key2)
    f2 = 8191 - (k2 & 8191)
    return f1 & (WW - 1), f2 & (WW - 1)


def _aligned(along, n0, n1):
    li = lax.broadcasted_iota(jnp.int32, (S, WW), 1)
    az = jnp.where((li == n0) | (li == n1), 0.0, along)
    mean = jnp.sum(az * li.astype(jnp.float32), axis=1, keepdims=True) / SEG
    return jnp.round(mean - mean[START:START + 1])


def _post_body(hist_ref, out_ref):
    ax_rows = []
    ay_rows = []
    for s in range(S):
        Hs = hist_ref[0, s]
        ax_rows.append(jnp.sum(Hs, axis=0).reshape(1, WW))
        ay_rows.append(jnp.sum(Hs, axis=1).reshape(1, HH))
    alongX = jnp.concatenate(ax_rows, axis=0)
    alongY = jnp.concatenate(ay_rows, axis=0)

    nx0, nx1 = _top2_noise(alongX)
    ny0, ny1 = _top2_noise(alongY)
    alX = _aligned(alongX, nx0, nx1)
    alY = _aligned(alongY, ny0, ny1)

    li = lax.broadcasted_iota(jnp.int32, (S, WW), 1)
    sx = jnp.sum(alongX.astype(jnp.int32) * li)
    sy = jnp.sum(alongY.astype(jnp.int32) * li)
    xd = jnp.floor(jnp.float32(WW // 2) - sx.astype(jnp.float32) / N)
    yd = jnp.floor(jnp.float32(HH // 2) - sy.astype(jnp.float32) / N)
    xd = xd.astype(jnp.int32)
    yd = yd.astype(jnp.int32)

    i0 = lax.broadcasted_iota(jnp.int32, (WW, WW), 0)
    i1 = lax.broadcasted_iota(jnp.int32, (WW, WW), 1)
    p0 = lax.broadcasted_iota(jnp.int32, (WW // 2, WW), 0)
    p1 = lax.broadcasted_iota(jnp.int32, (WW // 2, WW), 1)
    q0 = lax.broadcasted_iota(jnp.int32, (WW, WW // 2), 0)
    q1 = lax.broadcasted_iota(jnp.int32, (WW, WW // 2), 1)

    def shift_mats(ax, ay):
        MrT = (jnp.clip(i1 - ay, 0, WW - 1) == i0).astype(jnp.float32)
        Mc = (jnp.clip(i0 - ax, 0, WW - 1) == i1).astype(jnp.float32)
        return MrT, Mc

    def pool_mats(ax, ay):
        PMrT = (jnp.clip(p1 - ay, 0, WW - 1) // 2 == p0).astype(jnp.float32)
        PMc = (jnp.clip(q0 - ax, 0, WW - 1) // 2 == q1).astype(jnp.float32)
        return PMrT, PMc

    def occupancy(Hs, ax, ay):
        PMrT, PMc = pool_mats(ax, ay)
        pooled = _bdot(_bdot(PMrT, Hs), PMc)
        return (pooled > 0.0).astype(jnp.float32)

    def shifted_img(Hs, ax, ay):
        MrT, Mc = shift_mats(ax, ay)
        return _exact_rmul(_exact_lmul(MrT, Hs), Mc)

    ax0 = alX[START, 0].astype(jnp.int32)
    ay0 = alY[START, 0].astype(jnp.int32)
    H0 = hist_ref[0, START]
    container = shifted_img(H0, ax0, ay0)
    ver = occupancy(H0, ax0, ay0)
    old_cnt = jnp.sum(ver)

    seg_iota = lax.broadcasted_iota(jnp.int32, (S, 1), 0)

    def w_cond(carry):
        s, _, _, _, stopped = carry
        return jnp.logical_and(s < S, jnp.logical_not(stopped))

    def w_body(carry):
        s, container, ver, old_cnt, _ = carry
        ax = jnp.sum(jnp.where(seg_iota == s, alX, 0.0)).astype(jnp.int32)
        ay = jnp.sum(jnp.where(seg_iota == s, alY, 0.0)).astype(jnp.int32)
        Hs = hist_ref[0, s]
        occ = occupancy(Hs, ax, ay)
        un = jnp.maximum(ver, occ)
        new_cnt = jnp.sum(un)
        new_info = new_cnt - old_cnt
        stop_now = 10.0 * new_info < new_cnt
        keep = jnp.where(stop_now, 0.0, 1.0)
        container = container + keep * shifted_img(Hs, ax, ay)
        ver = ver + keep * (un - ver)
        old_cnt = old_cnt + keep * new_info
        return s + 1, container, ver, old_cnt, stop_now

    _, container, ver, old_cnt, _ = lax.while_loop(
        w_cond, w_body,
        (jnp.int32(START + 1), container, ver, old_cnt, jnp.bool_(False)))

    FrT = (i1 + yd == i0).astype(jnp.float32)
    Fc = (i0 + xd == i1).astype(jnp.float32)
    out_ref[0, 0] = _dot(_dot(FrT, container, lax.Precision.HIGHEST), Fc,
                         lax.Precision.HIGHEST)


def _post_tc(hist4):
    return pl.pallas_call(
        _post_body,
        grid=(B,),
        in_specs=[pl.BlockSpec((1, S, HH, WW), lambda b: (b, 0, 0, 0))],
        out_specs=pl.BlockSpec((1, 1, HH, WW), lambda b: (b, 0, 0, 0)),
        out_shape=jax.ShapeDtypeStruct((B, 1, HH, WW), jnp.float32),
    )(hist4)


def kernel(events):
    ev1 = events.reshape(PAIRS * ROW_W)
    hist = _hist_sc()(ev1)
    return _post_tc(hist.reshape(B, S, HH, WW))


# speedup vs baseline: 1.0314x; 1.0314x over previous
import functools

import jax
import jax.numpy as jnp
from jax import lax
from jax.experimental import pallas as pl
from jax.experimental.pallas import tpu as pltpu
from jax.experimental.pallas import tpu_sc as plsc

B = 4
N = 524288
S = 32
SEG = N // S
HH = 256
WW = 256
START = 2

NC = 2
NS = 16
NW = NC * NS
PAIRS = B * S
PER_W = PAIRS // NW

CHUNK = 4096
CH_W = CHUNK * 5
ROW_W = SEG * 5


def _hist_body(ev_hbm, out_hbm, hist_v, buf0, buf1, sem_a, sem_b, sem_out):
    cid = lax.axis_index("c")
    sid = lax.axis_index("s")
    wid = sid * NC + cid

    lane5 = jnp.arange(16, dtype=jnp.int32) * 5
    ones = jnp.ones((16,), jnp.float32)
    zeros = jnp.zeros((16,), jnp.float32)

    bufs = (buf0, buf1)
    sems = (sem_a, sem_b)
    nchunks = SEG // CHUNK
    out_cp = None
    for k in range(PER_W):
        p = wid * PER_W + k
        cps = [None, None]
        cps[0] = pltpu.async_copy(
            ev_hbm.at[p, pl.ds(0, CH_W)], bufs[0], sems[0])
        if out_cp is not None:
            out_cp.wait()

        def _zero(r, _):
            base = r * 256
            for u in range(16):
                hist_v[pl.ds(base + u * 16, 16)] = zeros
            return 0
        lax.fori_loop(0, (HH * WW) // 256, _zero, 0)

        for ci in range(nchunks):
            cur = ci % 2
            if ci + 1 < nchunks:
                nxt = (ci + 1) % 2
                cps[nxt] = pltpu.async_copy(
                    ev_hbm.at[p, pl.ds((ci + 1) * CH_W, CH_W)],
                    bufs[nxt], sems[nxt])
            cps[cur].wait()
            buf = bufs[cur]

            def _scat(j, _):
                base = j * 320
                for u in range(4):
                    off = lane5 + (base + u * 80)
                    x = plsc.load_gather(buf, [off])
                    y = plsc.load_gather(buf, [off + 1])
                    bins = (y * 256.0 + x).astype(jnp.int32)
                    plsc.addupdate_scatter(hist_v, [bins], ones)
                return 0
            lax.fori_loop(0, CHUNK // 64, _scat, 0)

        out_cp = pltpu.async_copy(hist_v, out_hbm.at[p], sem_out)
    out_cp.wait()


@functools.lru_cache(maxsize=1)
def _hist_sc():
    return pl.kernel(
        _hist_body,
        out_type=jax.ShapeDtypeStruct((PAIRS, HH * WW), jnp.float32),
        mesh=plsc.VectorSubcoreMesh(core_axis_name="c", subcore_axis_name="s"),
        compiler_params=pltpu.CompilerParams(needs_layout_passes=False),
        scratch_types=[
            pltpu.VMEM((HH * WW,), jnp.float32),
            pltpu.VMEM((CH_W,), jnp.float32),
            pltpu.VMEM((CH_W,), jnp.float32),
            pltpu.SemaphoreType.DMA,
            pltpu.SemaphoreType.DMA,
            pltpu.SemaphoreType.DMA,
        ],
    )


def _dot(a, b, prec=lax.Precision.DEFAULT):
    return lax.dot_general(
        a, b, (((1,), (0,)), ((), ())),
        precision=prec,
        preferred_element_type=jnp.float32)


def _bdot(a, b):
    return _dot(a.astype(jnp.bfloat16), b.astype(jnp.bfloat16))


def _split256(x):
    hi = jnp.floor(x * (1.0 / 256.0))
    lo = x - 256.0 * hi
    return hi.astype(jnp.bfloat16), lo.astype(jnp.bfloat16)


def _exact_lmul(m, x):
    hi, lo = _split256(x)
    mb = m.astype(jnp.bfloat16)
    return 256.0 * _dot(mb, hi) + _dot(mb, lo)


def _exact_rmul(x, m):
    hi, lo = _split256(x)
    mb = m.astype(jnp.bfloat16)
    return 256.0 * _dot(hi, mb) + _dot(lo, mb)


def _top2_noise(along):
    v = along.astype(jnp.int32)
    si = lax.broadcasted_iota(jnp.int32, (S, WW), 0)
    li = lax.broadcasted_iota(jnp.int32, (S, WW), 1)
    flat = si * WW + li
    key = v * 8192 + (8191 - flat)
    k1 = jnp.max(key)
    f1 = 8191 - (k1 & 8191)
    key2 = jnp.where(flat == f1, -1, key)
    k2 = jnp.max(key2)
    f2 = 8191 - (k2 & 8191)
    return f1 & (WW - 1), f2 & (WW - 1)


def _aligned(along, n0, n1):
    li = lax.broadcasted_iota(jnp.int32, (S, WW), 1)
    az = jnp.where((li == n0) | (li == n1), 0.0, along)
    mean = jnp.sum(az * li.astype(jnp.float32), axis=1, keepdims=True) / SEG
    return jnp.round(mean - mean[START:START + 1])


def _post_body(hist_ref, out_ref):
    ax_rows = []
    ay_rows = []
    for s in range(S):
        Hs = hist_ref[0, s]
        ax_rows.append(jnp.sum(Hs, axis=0).reshape(1, WW))
        ay_rows.append(jnp.sum(Hs, axis=1).reshape(1, HH))
    alongX = jnp.concatenate(ax_rows, axis=0)
    alongY = jnp.concatenate(ay_rows, axis=0)

    nx0, nx1 = _top2_noise(alongX)
    ny0, ny1 = _top2_noise(alongY)
    alX = _aligned(alongX, nx0, nx1)
    alY = _aligned(alongY, ny0, ny1)

    li = lax.broadcasted_iota(jnp.int32, (S, WW), 1)
    sx = jnp.sum(alongX.astype(jnp.int32) * li)
    sy = jnp.sum(alongY.astype(jnp.int32) * li)
    xd = jnp.floor(jnp.float32(WW // 2) - sx.astype(jnp.float32) / N)
    yd = jnp.floor(jnp.float32(HH // 2) - sy.astype(jnp.float32) / N)
    xd = xd.astype(jnp.int32)
    yd = yd.astype(jnp.int32)

    i0 = lax.broadcasted_iota(jnp.int32, (WW, WW), 0)
    i1 = lax.broadcasted_iota(jnp.int32, (WW, WW), 1)
    p0 = lax.broadcasted_iota(jnp.int32, (WW // 2, WW), 0)
    p1 = lax.broadcasted_iota(jnp.int32, (WW // 2, WW), 1)
    q0 = lax.broadcasted_iota(jnp.int32, (WW, WW // 2), 0)
    q1 = lax.broadcasted_iota(jnp.int32, (WW, WW // 2), 1)

    def shift_mats(ax, ay):
        MrT = (jnp.clip(i1 - ay, 0, WW - 1) == i0).astype(jnp.float32)
        Mc = (jnp.clip(i0 - ax, 0, WW - 1) == i1).astype(jnp.float32)
        return MrT, Mc

    def pool_mats(ax, ay):
        PMrT = (jnp.clip(p1 - ay, 0, WW - 1) // 2 == p0).astype(jnp.float32)
        PMc = (jnp.clip(q0 - ax, 0, WW - 1) // 2 == q1).astype(jnp.float32)
        return PMrT, PMc

    def occupancy(Hs, ax, ay):
        PMrT, PMc = pool_mats(ax, ay)
        pooled = _bdot(_bdot(PMrT, Hs), PMc)
        return (pooled > 0.0).astype(jnp.float32)

    def shifted_img(Hs, ax, ay):
        MrT, Mc = shift_mats(ax, ay)
        return _exact_rmul(_exact_lmul(MrT, Hs), Mc)

    ax0 = alX[START, 0].astype(jnp.int32)
    ay0 = alY[START, 0].astype(jnp.int32)
    H0 = hist_ref[0, START]
    container = shifted_img(H0, ax0, ay0)
    ver = occupancy(H0, ax0, ay0)
    old_cnt = jnp.sum(ver)

    seg_iota = lax.broadcasted_iota(jnp.int32, (S, 1), 0)

    def w_cond(carry):
        s, _, _, _, stopped = carry
        return jnp.logical_and(s < S, jnp.logical_not(stopped))

    def w_body(carry):
        s, container, ver, old_cnt, _ = carry
        ax = jnp.sum(jnp.where(seg_iota == s, alX, 0.0)).astype(jnp.int32)
        ay = jnp.sum(jnp.where(seg_iota == s, alY, 0.0)).astype(jnp.int32)
        Hs = hist_ref[0, s]
        occ = occupancy(Hs, ax, ay)
        un = jnp.maximum(ver, occ)
        new_cnt = jnp.sum(un)
        new_info = new_cnt - old_cnt
        stop_now = 10.0 * new_info < new_cnt
        keep = jnp.where(stop_now, 0.0, 1.0)
        container = container + keep * shifted_img(Hs, ax, ay)
        ver = ver + keep * (un - ver)
        old_cnt = old_cnt + keep * new_info
        return s + 1, container, ver, old_cnt, stop_now

    _, container, ver, old_cnt, _ = lax.while_loop(
        w_cond, w_body,
        (jnp.int32(START + 1), container, ver, old_cnt, jnp.bool_(False)))

    FrT = (i1 + yd == i0).astype(jnp.float32)
    Fc = (i0 + xd == i1).astype(jnp.float32)
    out_ref[0, 0] = _dot(_dot(FrT, container, lax.Precision.HIGHEST), Fc,
                         lax.Precision.HIGHEST)


def _post_tc(hist4):
    return pl.pallas_call(
        _post_body,
        grid=(B,),
        in_specs=[pl.BlockSpec((1, S, HH, WW), lambda b: (b, 0, 0, 0))],
        out_specs=pl.BlockSpec((1, 1, HH, WW), lambda b: (b, 0, 0, 0)),
        out_shape=jax.ShapeDtypeStruct((B, 1, HH, WW), jnp.float32),
    )(hist4)


def kernel(events):
    ev2 = events.reshape(PAIRS, ROW_W)
    hist = _hist_sc()(ev2)
    return _post_tc(hist.reshape(B, S, HH, WW))
